# Initial kernel scaffold; baseline (speedup 1.0000x reference)
import jax
import jax.numpy as jnp
from jax.experimental import pallas as pl


def kernel(x):
    raise NotImplementedError("write your pallas kernel here")


# speedup vs baseline: 1.9122x; 1.9122x over previous
import functools

import jax
import jax.numpy as jnp
from jax import lax
from jax.experimental import pallas as pl


def _roll0(x, s):
    return jnp.concatenate([x[s:, :], x[:s, :]], axis=0)


def _roll1(x, s, axis):
    if axis == 0:
        return jnp.concatenate([x[s:, :], x[:s, :]], axis=0)
    return jnp.concatenate([x[:, s:], x[:, :s]], axis=1)


def _bitonic_sort_rows_kernel(x_ref, o_ref, *, L, CH):
    N = 1 << L
    R = N // 128
    nchunks = R // CH

    def cp(c, _):
        cb = c * CH
        o_ref[0, pl.ds(cb, CH), :] = x_ref[0, pl.ds(cb, CH), :]
        return 0

    lax.fori_loop(0, nchunks, cp, 0)

    row_iota = lax.broadcasted_iota(jnp.int32, (CH, 1), 0)
    lane_iota = lax.broadcasted_iota(jnp.int32, (1, 128), 1)
    lch = CH.bit_length() - 1

    def emit_slab_stage(k, j):
        s_rows = 1 << (j - 7)
        ratio = s_rows // CH
        npair_chunks = (R // 2) // CH

        def body(m, _):
            if ratio > 1:
                g = m // ratio
                t = m - g * ratio
            else:
                g = m
                t = 0
            lo_base = g * (2 * s_rows) + t * CH
            hi_base = lo_base + s_rows
            lo = o_ref[0, pl.ds(lo_base, CH), :]
            hi = o_ref[0, pl.ds(hi_base, CH), :]
            mn = jnp.minimum(lo, hi)
            mx = jnp.maximum(lo, hi)
            if k == L:
                nlo, nhi = mn, mx
            else:
                p = k - 7
                asc = ((lo_base >> p) & 1) == 0
                nlo = jnp.where(asc, mn, mx)
                nhi = jnp.where(asc, mx, mn)
            o_ref[0, pl.ds(lo_base, CH), :] = nlo
            o_ref[0, pl.ds(hi_base, CH), :] = nhi
            return 0

        lax.fori_loop(0, npair_chunks, body, 0)

    def emit_roll_stage(k, j):
        if j >= 7:
            s = 1 << (j - 7)
            axis = 0
            lob = ((row_iota >> (j - 7)) & 1) == 0
            size = CH
        else:
            s = 1 << j
            axis = 1
            lob = ((lane_iota >> j) & 1) == 0
            size = 128

        def body(c, _):
            cb = c * CH
            x = o_ref[0, pl.ds(cb, CH), :]
            pm = _roll1(x, s, axis)
            pp = _roll1(x, size - s, axis)
            part = jnp.where(lob, pm, pp)
            mn = jnp.minimum(x, part)
            mx = jnp.maximum(x, part)
            if k == L:
                tm = lob
            elif k < 7:
                asc = ((lane_iota >> k) & 1) == 0
                tm = lob == asc
            else:
                p = k - 7
                if p < lch:
                    asc = ((row_iota >> p) & 1) == 0
                    tm = lob == asc
                else:
                    asc = ((cb >> p) & 1) == 0
                    tm = lob == asc
            o_ref[0, pl.ds(cb, CH), :] = jnp.where(tm, mn, mx)
            return 0

        lax.fori_loop(0, nchunks, body, 0)

    for k in range(1, L + 1):
        for j in range(k - 1, -1, -1):
            if j >= 7 and (1 << (j - 7)) >= CH:
                emit_slab_stage(k, j)
            else:
                emit_roll_stage(k, j)


def _sort_padded(x3, L, CH, interpret=False):
    B, R, _ = x3.shape
    return pl.pallas_call(
        functools.partial(_bitonic_sort_rows_kernel, L=L, CH=CH),
        grid=(B,),
        in_specs=[pl.BlockSpec((1, R, 128), lambda i: (i, 0, 0))],
        out_specs=pl.BlockSpec((1, R, 128), lambda i: (i, 0, 0)),
        out_shape=jax.ShapeDtypeStruct((B, R, 128), jnp.float32),
        interpret=interpret,
    )(x3)


def kernel(x):
    B, n = x.shape
    L = max(8, (n - 1).bit_length())
    N = 1 << L
    R = N // 128
    CH = min(256, R)
    xp = jnp.pad(x, ((0, 0), (0, N - n)), constant_values=jnp.float32(jnp.inf))
    out = _sort_padded(xp.reshape(B, R, 128), L, CH)
    return out.reshape(B, N)[:, :n]
